# Initial kernel scaffold; baseline (speedup 1.0000x reference)
import jax
import jax.numpy as jnp
from jax.experimental import pallas as pl


def kernel(node_features, mesh_edge_features, senders, receivers, enc_node_W1, enc_node_b1, enc_node_W2, enc_node_b2, enc_node_g, enc_node_be, enc_edge_W1, enc_edge_b1, enc_edge_W2, enc_edge_b2, enc_edge_g, enc_edge_be, blk_edge_W1, blk_edge_b1, blk_edge_W2, blk_edge_b2, blk_edge_g, blk_edge_be, blk_node_W1, blk_node_b1, blk_node_W2, blk_node_b2, blk_node_g, blk_node_be, dec_W1, dec_b1, dec_W2, dec_b2):
    raise NotImplementedError("write your pallas kernel here")


# speedup vs baseline: 2.8197x; 2.8197x over previous
import functools

import jax
import jax.numpy as jnp
from jax import lax
from jax.experimental import pallas as pl
from jax.experimental.pallas import tpu as pltpu
from jax.experimental.pallas import tpu_sc as plsc

_N = 10000
_E = 320000
_LAT = 128
_STEPS = 15
_F32 = jnp.float32


_GC = 80
_GK = 5
_EPT_G = _E // 16
_NCH_G = _EPT_G // _GC
_NHALF = 5120
_APAD = 5248

def _gather_body(p_hbm, q_hbm, s_hbm, r_hbm, gs_hbm, gq_hbm, idx_v, rows_v, sem):
    c = lax.axis_index("c")
    s = lax.axis_index("s")
    ebase = s * _EPT_G

    def side(idx_hbm, tab_hbm, out_hbm):
        pltpu.sync_copy(idx_hbm.at[s], idx_v)

        def outer(j, carry):
            hs = []
            for b in range(_GK):
                ch = j * _GK + b
                hs.append(pltpu.async_copy(tab_hbm.at[idx_v.at[ch]], rows_v.at[b], sem))
            for h in hs:
                h.wait()
            hs2 = []
            for b in range(_GK):
                ch = j * _GK + b
                hs2.append(pltpu.async_copy(
                    rows_v.at[b], out_hbm.at[pl.ds(ebase + ch * _GC, _GC)], sem))
            for h in hs2:
                h.wait()
            return carry

        lax.fori_loop(0, _NCH_G // _GK, outer, 0)

    @pl.when(c == 0)
    def _():
        side(s_hbm, p_hbm, gs_hbm)

    @pl.when(c == 1)
    def _():
        side(r_hbm, q_hbm, gq_hbm)


@functools.lru_cache(maxsize=None)
def _gather_kernel():
    mesh = plsc.VectorSubcoreMesh(core_axis_name="c", subcore_axis_name="s")
    return pl.kernel(
        _gather_body,
        mesh=mesh,
        out_type=(
            jax.ShapeDtypeStruct((_E, _LAT), _F32),
            jax.ShapeDtypeStruct((_E, _LAT), _F32),
        ),
        scratch_types=[
            pltpu.VMEM((_NCH_G, _GC), jnp.int32),
            pltpu.VMEM((_GK, _GC, _LAT), _F32),
            pltpu.SemaphoreType.DMA,
        ],
    )


def _gather(p, q, s2, r2):
    return _gather_kernel()(p, q, s2, r2)


def _scatter_body(e_hbm, r_hbm, out_hbm, idx_v, rows_v, zbuf, aggr_sh, sem):
    c = lax.axis_index("c")
    s = lax.axis_index("s")
    lo = c * _NHALF
    ebase = s * _EPT_G

    pltpu.sync_copy(r_hbm.at[s], idx_v)

    def loc(i, carry):
        r = i // (_GC // 16)
        col = (i % (_GC // 16)) * 16
        v = idx_v[r, pl.ds(col, 16)] - lo
        inb = (v >= 0) & (v < _NHALF)
        idx_v[r, pl.ds(col, 16)] = jnp.where(inb, v, _NHALF)
        return carry

    lax.fori_loop(0, _NCH_G * (_GC // 16), loc, 0)

    def zb(i, carry):
        zbuf[i // 8, pl.ds((i % 8) * 16, 16)] = jnp.zeros((16,), _F32)
        return carry

    lax.fori_loop(0, 64, zb, 0)

    a0 = s * (_APAD // 16)

    def zrow(k, carry):
        pltpu.sync_copy(zbuf, aggr_sh.at[pl.ds(a0 + k * 8, 8)])
        return carry

    lax.fori_loop(0, _APAD // 16 // 8, zrow, 0)
    plsc.subcore_barrier()

    def outer(j, carry):
        hs = []
        for b in range(_GK):
            ch = j * _GK + b
            hs.append(pltpu.async_copy(
                e_hbm.at[pl.ds(ebase + ch * _GC, _GC)], rows_v.at[b], sem))
        for h in hs:
            h.wait()
        for b in range(_GK):
            ch = j * _GK + b
            pltpu.sync_copy(rows_v.at[b], aggr_sh.at[idx_v.at[ch]], add=True)
        return carry

    lax.fori_loop(0, _NCH_G // _GK, outer, 0)
    plsc.subcore_barrier()

    o0 = s * (_NHALF // 16)
    for t in range(2):
        pltpu.sync_copy(aggr_sh.at[pl.ds(o0 + t * 160, 160)],
                        out_hbm.at[pl.ds(c * _NHALF + o0 + t * 160, 160)])


@functools.lru_cache(maxsize=None)
def _scatter_kernel():
    mesh = plsc.VectorSubcoreMesh(core_axis_name="c", subcore_axis_name="s")
    return pl.kernel(
        _scatter_body,
        mesh=mesh,
        out_type=jax.ShapeDtypeStruct((2 * _NHALF, _LAT), _F32),
        scratch_types=[
            pltpu.VMEM((_NCH_G, _GC), jnp.int32),
            pltpu.VMEM((_GK, _GC, _LAT), _F32),
            pltpu.VMEM((8, _LAT), _F32),
            pltpu.VMEM_SHARED((_APAD, _LAT), _F32),
            pltpu.SemaphoreType.DMA,
        ],
    )


def _scatter(ne, r3):
    return _scatter_kernel()(ne, r3)


def _ln(h, g, be):
    m = jnp.mean(h, axis=-1, keepdims=True)
    d = h - m
    v = jnp.mean(d * d, axis=-1, keepdims=True)
    return d * lax.rsqrt(v + 1e-5) * g + be


def _vec_spec():
    return pl.BlockSpec((_LAT,), lambda i: (0,))


def _mat_spec(r):
    return pl.BlockSpec((r, _LAT), lambda i: (0, 0))


def _row_spec(bn, w):
    return pl.BlockSpec((bn, w), lambda i: (i, 0))


_PD = lax.Precision.DEFAULT
_PH = lax.Precision.HIGHEST
_P_DEC = (_PD, _PH)


def _dot(a, b, prec=_PD):
    return jnp.dot(a, b, preferred_element_type=_F32, precision=prec)


def _enc_body(x_ref, w1, b1, w2, b2, g, be, nl_ref):
    h = jnp.maximum(_dot(x_ref[...], w1[...]) + b1[...], 0.0)
    h = jnp.maximum(_dot(h, w2[...]) + b2[...], 0.0)
    nl_ref[...] = _ln(h, g[...], be[...])


def _enc_call(x, w1, b1, w2, b2, g, be, rows, fw):
    bn = 2000
    return pl.pallas_call(
        _enc_body,
        grid=(rows // bn,),
        in_specs=[_row_spec(bn, fw), pl.BlockSpec((fw, _LAT), lambda i: (0, 0)),
                  _vec_spec(), _mat_spec(_LAT), _vec_spec(), _vec_spec(), _vec_spec()],
        out_specs=_row_spec(bn, _LAT),
        out_shape=jax.ShapeDtypeStruct((rows, _LAT), _F32),
    )(x, w1, b1, w2, b2, g, be)


def _edge_body(gs_ref, gq_ref, el_ref, w1, b1, w2, b2, g, be, elo_ref, ne_ref):
    el = el_ref[...]
    x = jnp.concatenate([gs_ref[...], gq_ref[...], el], axis=-1)
    h = jnp.maximum(_dot(x, w1[...]) + b1[...], 0.0)
    h = jnp.maximum(_dot(h, w2[...]) + b2[...], 0.0)
    ne = _ln(h, g[...], be[...])
    ne_ref[...] = ne
    elo_ref[...] = el + ne


def _edge_call(gs, gq, el, w1, b1, w2, b2, g, be):
    bn = 2000
    return pl.pallas_call(
        _edge_body,
        grid=(_E // bn,),
        in_specs=[_row_spec(bn, _LAT)] * 3 + [_mat_spec(3 * _LAT), _vec_spec(),
                  _mat_spec(_LAT), _vec_spec(), _vec_spec(), _vec_spec()],
        out_specs=[_row_spec(bn, _LAT)] * 2,
        out_shape=[jax.ShapeDtypeStruct((_E, _LAT), _F32)] * 2,
    )(gs, gq, el, w1, b1, w2, b2, g, be)


def _node_body(nl_ref, ag_ref, w1, b1, w2, b2, g, be, nlo_ref):
    nl = nl_ref[...]
    x = jnp.concatenate([nl, ag_ref[...]], axis=-1)
    h = jnp.maximum(_dot(x, w1[...]) + b1[...], 0.0)
    h = jnp.maximum(_dot(h, w2[...]) + b2[...], 0.0)
    nlo_ref[...] = nl + _ln(h, g[...], be[...])


def _node_call(nl, ag, w1, b1, w2, b2, g, be):
    bn = 2000
    return pl.pallas_call(
        _node_body,
        grid=(_N // bn,),
        in_specs=[_row_spec(bn, _LAT)] * 2 + [_mat_spec(2 * _LAT), _vec_spec(),
                  _mat_spec(_LAT), _vec_spec(), _vec_spec(), _vec_spec()],
        out_specs=_row_spec(bn, _LAT),
        out_shape=jax.ShapeDtypeStruct((_N, _LAT), _F32),
    )(nl, ag, w1, b1, w2, b2, g, be)


def _dec_body(nl_ref, w1, b1, w2, b2, dt, out_ref):
    h = _dot(nl_ref[...], w1[...], _P_DEC[0]) + b1[...]
    h = h * jax.nn.sigmoid(h)
    out_ref[...] = (_dot(h, w2[...], _P_DEC[1]) + b2[...]) * dt[...]


def _dec_call(nl, w1, b1, w2, b2, dt):
    bn = 2000
    return pl.pallas_call(
        _dec_body,
        grid=(_N // bn,),
        in_specs=[_row_spec(bn, _LAT), _mat_spec(_LAT), _vec_spec(),
                  _mat_spec(_LAT), _vec_spec(), _vec_spec()],
        out_specs=_row_spec(bn, _LAT),
        out_shape=jax.ShapeDtypeStruct((_N, _LAT), _F32),
    )(nl, w1, b1, w2, b2, dt)


def kernel(node_features, mesh_edge_features, senders, receivers,
           enc_node_W1, enc_node_b1, enc_node_W2, enc_node_b2, enc_node_g, enc_node_be,
           enc_edge_W1, enc_edge_b1, enc_edge_W2, enc_edge_b2, enc_edge_g, enc_edge_be,
           blk_edge_W1, blk_edge_b1, blk_edge_W2, blk_edge_b2, blk_edge_g, blk_edge_be,
           blk_node_W1, blk_node_b1, blk_node_W2, blk_node_b2, blk_node_g, blk_node_be,
           dec_W1, dec_b1, dec_W2, dec_b2):
    s3g = senders.astype(jnp.int32).reshape(16, _NCH_G, _GC)
    r3g = receivers.astype(jnp.int32).reshape(16, _NCH_G, _GC)

    xn = jnp.pad(node_features.astype(_F32), ((0, 0), (0, 4)))
    w1n = jnp.pad(enc_node_W1.astype(_F32), ((0, 4), (0, 0)))
    xe = jnp.pad(mesh_edge_features.astype(_F32), ((0, 0), (0, 4)))
    w1e = jnp.pad(enc_edge_W1.astype(_F32), ((0, 4), (0, 0)))

    node_lat = _enc_call(xn, w1n, enc_node_b1, enc_node_W2, enc_node_b2,
                         enc_node_g, enc_node_be, _N, 16)
    edge_lat = _enc_call(xe, w1e, enc_edge_b1, enc_edge_W2, enc_edge_b2,
                         enc_edge_g, enc_edge_be, _E, 8)

    for i in range(_STEPS):
        gs, gq = _gather(node_lat, node_lat, s3g, r3g)
        edge_lat, new_e = _edge_call(gs, gq, edge_lat, blk_edge_W1[i], blk_edge_b1[i],
                                     blk_edge_W2[i], blk_edge_b2[i],
                                     blk_edge_g[i], blk_edge_be[i])
        aggr = _scatter(new_e, r3g)
        node_lat = _node_call(node_lat, aggr[:_N],
                              blk_node_W1[i], blk_node_b1[i],
                              blk_node_W2[i], blk_node_b2[i],
                              blk_node_g[i], blk_node_be[i])

    w1d = jnp.pad(dec_W1.astype(_F32), ((0, 0), (0, 120)))
    b1d = jnp.pad(dec_b1.astype(_F32), (0, 120))
    w2d = jnp.pad(dec_W2.astype(_F32), ((0, 120), (0, 113)))
    b2d = jnp.pad(dec_b2.astype(_F32), (0, 113))
    dt = jnp.pad(jnp.repeat(jnp.arange(1, 6), 3).astype(_F32), (0, 113))
    dec = _dec_call(node_lat, w1d, b1d, w2d, b2d, dt)
    return dec[:, :15].reshape(_N, 5, 3).transpose(1, 0, 2)


# speedup vs baseline: 2.9999x; 1.0639x over previous
import functools

import jax
import jax.numpy as jnp
from jax import lax
from jax.experimental import pallas as pl
from jax.experimental.pallas import tpu as pltpu
from jax.experimental.pallas import tpu_sc as plsc

_N = 10000
_E = 320000
_LAT = 128
_STEPS = 15
_F32 = jnp.float32


_GC = 80
_GK = 5
_EPT_G = _E // 16
_NCH_G = _EPT_G // _GC
_NHALF = 5120
_APAD = 5248

def _gather_body(p_hbm, q_hbm, s_hbm, r_hbm, gs_hbm, gq_hbm, idx_v, rows_v, sem, sem2):
    c = lax.axis_index("c")
    s = lax.axis_index("s")
    ebase = s * _EPT_G

    def side(idx_hbm, tab_hbm, out_hbm):
        pltpu.sync_copy(idx_hbm.at[s], idx_v)

        def outer(j, carry):
            hs = []
            for b in range(_GK):
                ch = j * _GK + b
                hs.append(pltpu.async_copy(tab_hbm.at[idx_v.at[ch]], rows_v.at[b], sem))
            hs2 = []
            for b in range(_GK):
                ch = j * _GK + b
                hs[b].wait()
                hs2.append(pltpu.async_copy(
                    rows_v.at[b], out_hbm.at[pl.ds(ebase + ch * _GC, _GC)], sem2))
            for h in hs2:
                h.wait()
            return carry

        lax.fori_loop(0, _NCH_G // _GK, outer, 0)

    @pl.when(c == 0)
    def _():
        side(s_hbm, p_hbm, gs_hbm)

    @pl.when(c == 1)
    def _():
        side(r_hbm, q_hbm, gq_hbm)


@functools.lru_cache(maxsize=None)
def _gather_kernel():
    mesh = plsc.VectorSubcoreMesh(core_axis_name="c", subcore_axis_name="s")
    return pl.kernel(
        _gather_body,
        mesh=mesh,
        out_type=(
            jax.ShapeDtypeStruct((_E, _LAT), _F32),
            jax.ShapeDtypeStruct((_E, _LAT), _F32),
        ),
        scratch_types=[
            pltpu.VMEM((_NCH_G, _GC), jnp.int32),
            pltpu.VMEM((_GK, _GC, _LAT), _F32),
            pltpu.SemaphoreType.DMA,
            pltpu.SemaphoreType.DMA,
        ],
    )


def _gather(p, q, s2, r2):
    return _gather_kernel()(p, q, s2, r2)


def _scatter_body(e_hbm, r_hbm, out_hbm, idx_v, rows_v, zbuf, aggr_sh, sem, sem2):
    c = lax.axis_index("c")
    s = lax.axis_index("s")
    lo = c * _NHALF
    ebase = s * _EPT_G

    pltpu.sync_copy(r_hbm.at[s], idx_v)

    def loc(i, carry):
        r = i // (_GC // 16)
        col = (i % (_GC // 16)) * 16
        v = idx_v[r, pl.ds(col, 16)] - lo
        inb = (v >= 0) & (v < _NHALF)
        idx_v[r, pl.ds(col, 16)] = jnp.where(inb, v, _NHALF)
        return carry

    lax.fori_loop(0, _NCH_G * (_GC // 16), loc, 0)

    def zb(i, carry):
        zbuf[i // 8, pl.ds((i % 8) * 16, 16)] = jnp.zeros((16,), _F32)
        return carry

    lax.fori_loop(0, 64, zb, 0)

    a0 = s * (_APAD // 16)

    def zrow(k, carry):
        pltpu.sync_copy(zbuf, aggr_sh.at[pl.ds(a0 + k * 8, 8)])
        return carry

    lax.fori_loop(0, _APAD // 16 // 8, zrow, 0)
    plsc.subcore_barrier()

    def outer(j, carry):
        hs = []
        for b in range(_GK):
            ch = j * _GK + b
            hs.append(pltpu.async_copy(
                e_hbm.at[pl.ds(ebase + ch * _GC, _GC)], rows_v.at[b], sem))
        hs2 = []
        for b in range(_GK):
            ch = j * _GK + b
            hs[b].wait()
            hs2.append(pltpu.async_copy(
                rows_v.at[b], aggr_sh.at[idx_v.at[ch]], sem2, add=True))
        for h in hs2:
            h.wait()
        return carry

    lax.fori_loop(0, _NCH_G // _GK, outer, 0)
    plsc.subcore_barrier()

    o0 = s * (_NHALF // 16)
    for t in range(2):
        pltpu.sync_copy(aggr_sh.at[pl.ds(o0 + t * 160, 160)],
                        out_hbm.at[pl.ds(c * _NHALF + o0 + t * 160, 160)])


@functools.lru_cache(maxsize=None)
def _scatter_kernel():
    mesh = plsc.VectorSubcoreMesh(core_axis_name="c", subcore_axis_name="s")
    return pl.kernel(
        _scatter_body,
        mesh=mesh,
        out_type=jax.ShapeDtypeStruct((2 * _NHALF, _LAT), _F32),
        scratch_types=[
            pltpu.VMEM((_NCH_G, _GC), jnp.int32),
            pltpu.VMEM((_GK, _GC, _LAT), _F32),
            pltpu.VMEM((8, _LAT), _F32),
            pltpu.VMEM_SHARED((_APAD, _LAT), _F32),
            pltpu.SemaphoreType.DMA,
            pltpu.SemaphoreType.DMA,
        ],
    )


def _scatter(ne, r3):
    return _scatter_kernel()(ne, r3)


def _ln(h, g, be):
    m = jnp.mean(h, axis=-1, keepdims=True)
    d = h - m
    v = jnp.mean(d * d, axis=-1, keepdims=True)
    return d * lax.rsqrt(v + 1e-5) * g + be


def _vec_spec():
    return pl.BlockSpec((_LAT,), lambda i: (0,))


def _mat_spec(r):
    return pl.BlockSpec((r, _LAT), lambda i: (0, 0))


def _row_spec(bn, w):
    return pl.BlockSpec((bn, w), lambda i: (i, 0))


_PD = lax.Precision.DEFAULT
_PH = lax.Precision.HIGHEST
_P_DEC = (_PD, _PH)


def _dot(a, b, prec=_PD):
    return jnp.dot(a, b, preferred_element_type=_F32, precision=prec)


def _enc_body(x_ref, w1, b1, w2, b2, g, be, nl_ref):
    h = jnp.maximum(_dot(x_ref[...], w1[...]) + b1[...], 0.0)
    h = jnp.maximum(_dot(h, w2[...]) + b2[...], 0.0)
    nl_ref[...] = _ln(h, g[...], be[...])


def _enc_call(x, w1, b1, w2, b2, g, be, rows, fw):
    bn = 2000
    return pl.pallas_call(
        _enc_body,
        grid=(rows // bn,),
        in_specs=[_row_spec(bn, fw), pl.BlockSpec((fw, _LAT), lambda i: (0, 0)),
                  _vec_spec(), _mat_spec(_LAT), _vec_spec(), _vec_spec(), _vec_spec()],
        out_specs=_row_spec(bn, _LAT),
        out_shape=jax.ShapeDtypeStruct((rows, _LAT), _F32),
    )(x, w1, b1, w2, b2, g, be)


def _edge_body(gs_ref, gq_ref, el_ref, w1, b1, w2, b2, g, be, elo_ref, ne_ref):
    el = el_ref[...]
    x = jnp.concatenate([gs_ref[...], gq_ref[...], el], axis=-1)
    h = jnp.maximum(_dot(x, w1[...]) + b1[...], 0.0)
    h = jnp.maximum(_dot(h, w2[...]) + b2[...], 0.0)
    ne = _ln(h, g[...], be[...])
    ne_ref[...] = ne
    elo_ref[...] = el + ne


def _edge_call(gs, gq, el, w1, b1, w2, b2, g, be):
    bn = 2000
    return pl.pallas_call(
        _edge_body,
        grid=(_E // bn,),
        in_specs=[_row_spec(bn, _LAT)] * 3 + [_mat_spec(3 * _LAT), _vec_spec(),
                  _mat_spec(_LAT), _vec_spec(), _vec_spec(), _vec_spec()],
        out_specs=[_row_spec(bn, _LAT)] * 2,
        out_shape=[jax.ShapeDtypeStruct((_E, _LAT), _F32)] * 2,
    )(gs, gq, el, w1, b1, w2, b2, g, be)


def _node_body(nl_ref, ag_ref, w1, b1, w2, b2, g, be, nlo_ref):
    nl = nl_ref[...]
    x = jnp.concatenate([nl, ag_ref[...]], axis=-1)
    h = jnp.maximum(_dot(x, w1[...]) + b1[...], 0.0)
    h = jnp.maximum(_dot(h, w2[...]) + b2[...], 0.0)
    nlo_ref[...] = nl + _ln(h, g[...], be[...])


def _node_call(nl, ag, w1, b1, w2, b2, g, be):
    bn = 2000
    return pl.pallas_call(
        _node_body,
        grid=(_N // bn,),
        in_specs=[_row_spec(bn, _LAT)] * 2 + [_mat_spec(2 * _LAT), _vec_spec(),
                  _mat_spec(_LAT), _vec_spec(), _vec_spec(), _vec_spec()],
        out_specs=_row_spec(bn, _LAT),
        out_shape=jax.ShapeDtypeStruct((_N, _LAT), _F32),
    )(nl, ag, w1, b1, w2, b2, g, be)


def _dec_body(nl_ref, w1, b1, w2, b2, dt, out_ref):
    h = _dot(nl_ref[...], w1[...], _P_DEC[0]) + b1[...]
    h = h * jax.nn.sigmoid(h)
    out_ref[...] = (_dot(h, w2[...], _P_DEC[1]) + b2[...]) * dt[...]


def _dec_call(nl, w1, b1, w2, b2, dt):
    bn = 2000
    return pl.pallas_call(
        _dec_body,
        grid=(_N // bn,),
        in_specs=[_row_spec(bn, _LAT), _mat_spec(_LAT), _vec_spec(),
                  _mat_spec(_LAT), _vec_spec(), _vec_spec()],
        out_specs=_row_spec(bn, _LAT),
        out_shape=jax.ShapeDtypeStruct((_N, _LAT), _F32),
    )(nl, w1, b1, w2, b2, dt)


def kernel(node_features, mesh_edge_features, senders, receivers,
           enc_node_W1, enc_node_b1, enc_node_W2, enc_node_b2, enc_node_g, enc_node_be,
           enc_edge_W1, enc_edge_b1, enc_edge_W2, enc_edge_b2, enc_edge_g, enc_edge_be,
           blk_edge_W1, blk_edge_b1, blk_edge_W2, blk_edge_b2, blk_edge_g, blk_edge_be,
           blk_node_W1, blk_node_b1, blk_node_W2, blk_node_b2, blk_node_g, blk_node_be,
           dec_W1, dec_b1, dec_W2, dec_b2):
    s3g = senders.astype(jnp.int32).reshape(16, _NCH_G, _GC)
    r3g = receivers.astype(jnp.int32).reshape(16, _NCH_G, _GC)

    xn = jnp.pad(node_features.astype(_F32), ((0, 0), (0, 4)))
    w1n = jnp.pad(enc_node_W1.astype(_F32), ((0, 4), (0, 0)))
    xe = jnp.pad(mesh_edge_features.astype(_F32), ((0, 0), (0, 4)))
    w1e = jnp.pad(enc_edge_W1.astype(_F32), ((0, 4), (0, 0)))

    node_lat = _enc_call(xn, w1n, enc_node_b1, enc_node_W2, enc_node_b2,
                         enc_node_g, enc_node_be, _N, 16)
    edge_lat = _enc_call(xe, w1e, enc_edge_b1, enc_edge_W2, enc_edge_b2,
                         enc_edge_g, enc_edge_be, _E, 8)

    for i in range(_STEPS):
        gs, gq = _gather(node_lat, node_lat, s3g, r3g)
        edge_lat, new_e = _edge_call(gs, gq, edge_lat, blk_edge_W1[i], blk_edge_b1[i],
                                     blk_edge_W2[i], blk_edge_b2[i],
                                     blk_edge_g[i], blk_edge_be[i])
        aggr = _scatter(new_e, r3g)
        node_lat = _node_call(node_lat, aggr[:_N],
                              blk_node_W1[i], blk_node_b1[i],
                              blk_node_W2[i], blk_node_b2[i],
                              blk_node_g[i], blk_node_be[i])

    w1d = jnp.pad(dec_W1.astype(_F32), ((0, 0), (0, 120)))
    b1d = jnp.pad(dec_b1.astype(_F32), (0, 120))
    w2d = jnp.pad(dec_W2.astype(_F32), ((0, 120), (0, 113)))
    b2d = jnp.pad(dec_b2.astype(_F32), (0, 113))
    dt = jnp.pad(jnp.repeat(jnp.arange(1, 6), 3).astype(_F32), (0, 113))
    dec = _dec_call(node_lat, w1d, b1d, w2d, b2d, dt)
    return dec[:, :15].reshape(_N, 5, 3).transpose(1, 0, 2)
